# baseline (speedup 1.0000x reference)
import functools

import jax
import jax.numpy as jnp
from jax import lax
from jax.experimental import pallas as pl
from jax.experimental.pallas import tpu as pltpu
from jax.experimental.pallas import tpu_sc as plsc

CTX_DIM = 512
N_CLS_CTX = 4
N_PRE = 5
TOK_LEN = 77
N_SUF = TOK_LEN - N_PRE - N_CLS_CTX
LANES = 16
CHUNK = 8
SPLIT = 16
NBUF = 4


def _sc_head(table3d, label, template, b):
    info = plsc.get_sparse_core_info()
    num_workers = info.num_cores * info.num_subcores
    assert b % num_workers == 0
    bpw = b // num_workers
    assert bpw % CHUNK == 0 and CHUNK % NBUF == 0
    n_chunks = bpw // CHUNK
    lane_steps = CTX_DIM // LANES

    mesh = plsc.VectorSubcoreMesh(core_axis_name="c", subcore_axis_name="s")

    @functools.partial(
        pl.kernel,
        mesh=mesh,
        out_type=jax.ShapeDtypeStruct((b, TOK_LEN, CTX_DIM), jnp.float32),
        scratch_types=[
            pltpu.VMEM((bpw,), jnp.int32),
            pltpu.VMEM((CHUNK, N_CLS_CTX, CTX_DIM), jnp.float32),
            pltpu.VMEM((CHUNK, N_CLS_CTX, CTX_DIM), jnp.float32),
        ] + [pltpu.VMEM((SPLIT, CTX_DIM), jnp.float32)] * NBUF + [
            pltpu.SemaphoreType.DMA,
            pltpu.SemaphoreType.DMA,
        ] + [pltpu.SemaphoreType.DMA] * NBUF + [
            pltpu.SemaphoreType.DMA,
        ],
    )
    def body(table_hbm, idx_hbm, tmpl_hbm, out_hbm,
             idx_v, rga, rgb, blk0, blk1, blk2, blk3,
             gsema, gsemb, osem0, osem1, osem2, osem3, tsem):
        blks = (blk0, blk1, blk2, blk3)
        osems = (osem0, osem1, osem2, osem3)
        wid = lax.axis_index("s") * info.num_cores + lax.axis_index("c")
        base = wid * bpw
        pltpu.sync_copy(idx_hbm.at[pl.ds(base, bpw)], idx_v)
        tmpl_cps = [
            pltpu.make_async_copy(tmpl_hbm.at[pl.ds(0, SPLIT)], blk, tsem)
            for blk in blks
        ]
        for cp in tmpl_cps:
            cp.start()
        for cp in tmpl_cps:
            cp.wait()

        gather_bufs = (rga, rgb)
        gather_sems = (gsema, gsemb)

        def start_gather(c):
            pltpu.make_async_copy(
                table_hbm.at[idx_v.at[pl.ds(c * CHUNK, CHUNK)]],
                gather_bufs[c % 2], gather_sems[c % 2]).start()

        start_gather(0)
        for c in range(n_chunks):
            rg = gather_bufs[c % 2]
            pltpu.make_async_copy(
                table_hbm.at[idx_v.at[pl.ds(c * CHUNK, CHUNK)]],
                rg, gather_sems[c % 2]).wait()
            if c + 1 < n_chunks:
                start_gather(c + 1)

            def do_quad(t, _):
                for i in range(NBUF):
                    k = NBUF * t + i
                    j = c * CHUNK + k

                    @pl.when(j >= NBUF)
                    def _wait_prev():
                        pltpu.make_async_copy(
                            blks[i], out_hbm.at[base, pl.ds(0, SPLIT)],
                            osems[i]).wait()

                    for r in range(N_CLS_CTX):
                        for cc in range(lane_steps):
                            sl = pl.ds(cc * LANES, LANES)
                            blks[i][N_PRE + r, sl] = rg[k, r, sl]
                    pltpu.make_async_copy(
                        blks[i], out_hbm.at[base + j, pl.ds(0, SPLIT)],
                        osems[i]).start()
                return _

            lax.fori_loop(0, CHUNK // NBUF, do_quad, 0)

        for i in range(NBUF):
            pltpu.make_async_copy(
                blks[i], out_hbm.at[base, pl.ds(0, SPLIT)], osems[i]).wait()

    return body(table3d, label, template)


def _tc_tail(buf, token_suffix, group=32):
    b = buf.shape[0]
    assert b % group == 0
    n_groups = b // group
    tail_rows = TOK_LEN - SPLIT
    suf_off = SPLIT - N_PRE - N_CLS_CTX

    nsem = 8

    def body(buf_ref, suf_ref, out_ref, suf_v, *sems):
        del buf_ref
        suf_v[...] = jnp.broadcast_to(
            suf_ref[:, suf_off:N_SUF, :], (group, tail_rows, CTX_DIM))
        copies = [
            pltpu.make_async_copy(
                suf_v,
                out_ref.at[pl.ds(i * group, group), pl.ds(SPLIT, tail_rows)],
                sems[i % nsem])
            for i in range(n_groups)
        ]
        for cp in copies:
            cp.start()
        for cp in copies:
            cp.wait()

    return pl.pallas_call(
        body,
        in_specs=[
            pl.BlockSpec(memory_space=pl.ANY),
            pl.BlockSpec((1, N_SUF, CTX_DIM), lambda: (0, 0, 0)),
        ],
        out_specs=pl.BlockSpec(memory_space=pl.ANY),
        out_shape=jax.ShapeDtypeStruct((b, TOK_LEN, CTX_DIM), jnp.float32),
        scratch_shapes=[pltpu.VMEM((group, tail_rows, CTX_DIM), jnp.float32)]
        + [pltpu.SemaphoreType.DMA] * nsem,
        input_output_aliases={0: 0},
    )(buf, token_suffix)


def kernel(label, cls_ctx, token_prefix, token_suffix):
    b = label.shape[0]
    template = jnp.zeros((SPLIT, CTX_DIM), jnp.float32)
    template = template.at[0:N_PRE].set(token_prefix[0])
    template = template.at[N_PRE + N_CLS_CTX:SPLIT].set(
        token_suffix[0, :SPLIT - N_PRE - N_CLS_CTX])
    buf = _sc_head(cls_ctx, label.astype(jnp.int32), template, b)
    return _tc_tail(buf, token_suffix)


# speedup vs baseline: 1.0804x; 1.0804x over previous
import functools

import jax
import jax.numpy as jnp
from jax import lax
from jax.experimental import pallas as pl
from jax.experimental.pallas import tpu as pltpu
from jax.experimental.pallas import tpu_sc as plsc

CTX_DIM = 512
N_CLS_CTX = 4
N_PRE = 5
TOK_LEN = 77
N_SUF = TOK_LEN - N_PRE - N_CLS_CTX
BLOCK_B = 64


def _sc_gather(table3d, label):
    info = plsc.get_sparse_core_info()
    num_workers = info.num_cores * info.num_subcores
    n = label.shape[0]
    assert n % num_workers == 0
    bpw = n // num_workers

    mesh = plsc.VectorSubcoreMesh(core_axis_name="c", subcore_axis_name="s")

    @functools.partial(
        pl.kernel,
        mesh=mesh,
        out_type=jax.ShapeDtypeStruct((n, N_CLS_CTX, CTX_DIM), jnp.float32),
        scratch_types=[
            pltpu.VMEM((bpw,), jnp.int32),
            pltpu.VMEM((bpw, N_CLS_CTX, CTX_DIM), jnp.float32),
            pltpu.SemaphoreType.DMA,
        ],
    )
    def gather_kernel(table_hbm, idx_hbm, out_hbm, idx_v, rows_v, sem):
        wid = lax.axis_index("s") * info.num_cores + lax.axis_index("c")
        base = wid * bpw
        pltpu.sync_copy(idx_hbm.at[pl.ds(base, bpw)], idx_v)
        pltpu.async_copy(table_hbm.at[idx_v], rows_v, sem).wait()
        pltpu.sync_copy(rows_v, out_hbm.at[pl.ds(base, bpw)])

    return gather_kernel(table3d, label)


def _tc_concat_half(buf, cls3, token_prefix, token_suffix, b, block_off):
    n = cls3.shape[0]
    assert n % BLOCK_B == 0

    def body(*refs):
        if buf is None:
            pref_ref, suf_ref, cls_ref, out_ref = refs
        else:
            _, pref_ref, suf_ref, cls_ref, out_ref = refs
        out_ref[:, 0:N_PRE, :] = jnp.broadcast_to(
            pref_ref[...], (BLOCK_B, N_PRE, CTX_DIM))
        out_ref[:, N_PRE:N_PRE + N_CLS_CTX, :] = cls_ref[...]
        out_ref[:, N_PRE + N_CLS_CTX:TOK_LEN, :] = jnp.broadcast_to(
            suf_ref[...], (BLOCK_B, N_SUF, CTX_DIM))

    in_specs = [
        pl.BlockSpec((1, N_PRE, CTX_DIM), lambda i: (0, 0, 0)),
        pl.BlockSpec((1, N_SUF, CTX_DIM), lambda i: (0, 0, 0)),
        pl.BlockSpec((BLOCK_B, N_CLS_CTX, CTX_DIM), lambda i: (i, 0, 0)),
    ]
    args = (token_prefix, token_suffix, cls3)
    aliases = {}
    if buf is not None:
        in_specs = [pl.BlockSpec(memory_space=pl.ANY)] + in_specs
        args = (buf,) + args
        aliases = {0: 0}

    return pl.pallas_call(
        body,
        grid=(n // BLOCK_B,),
        in_specs=in_specs,
        out_specs=pl.BlockSpec(
            (BLOCK_B, TOK_LEN, CTX_DIM),
            lambda i: (i + block_off, 0, 0)),
        out_shape=jax.ShapeDtypeStruct((b, TOK_LEN, CTX_DIM), jnp.float32),
        input_output_aliases=aliases,
    )(*args)


def kernel(label, cls_ctx, token_prefix, token_suffix):
    b = label.shape[0]
    half = b // 2
    label32 = label.astype(jnp.int32)
    cls_a = _sc_gather(cls_ctx, label32[:half])
    cls_b = _sc_gather(cls_ctx, label32[half:])
    buf = _tc_concat_half(None, cls_a, token_prefix, token_suffix, b, 0)
    return _tc_concat_half(buf, cls_b, token_prefix, token_suffix, b,
                           half // BLOCK_B)


# speedup vs baseline: 1.0924x; 1.0111x over previous
import functools

import jax
import jax.numpy as jnp
from jax import lax
from jax.experimental import pallas as pl
from jax.experimental.pallas import tpu as pltpu
from jax.experimental.pallas import tpu_sc as plsc

CTX_DIM = 512
N_CLS_CTX = 4
N_PRE = 5
TOK_LEN = 77
N_SUF = TOK_LEN - N_PRE - N_CLS_CTX


def _sc_gather(table3d, label):
    info = plsc.get_sparse_core_info()
    num_workers = info.num_cores * info.num_subcores
    b = label.shape[0]
    assert b % num_workers == 0
    bpw = b // num_workers

    mesh = plsc.VectorSubcoreMesh(core_axis_name="c", subcore_axis_name="s")

    @functools.partial(
        pl.kernel,
        mesh=mesh,
        out_type=jax.ShapeDtypeStruct((b, N_CLS_CTX, CTX_DIM), jnp.float32),
        scratch_types=[
            pltpu.VMEM((bpw,), jnp.int32),
            pltpu.VMEM((bpw, N_CLS_CTX, CTX_DIM), jnp.float32),
            pltpu.SemaphoreType.DMA,
        ],
    )
    def gather_kernel(table_hbm, idx_hbm, out_hbm, idx_v, rows_v, sem):
        wid = lax.axis_index("s") * info.num_cores + lax.axis_index("c")
        base = wid * bpw
        pltpu.sync_copy(idx_hbm.at[pl.ds(base, bpw)], idx_v)
        pltpu.async_copy(table_hbm.at[idx_v], rows_v, sem).wait()
        pltpu.sync_copy(rows_v, out_hbm.at[pl.ds(base, bpw)])

    return gather_kernel(table3d, label)


def _tc_concat(cls3, token_prefix, token_suffix, block_b=64):
    b = cls3.shape[0]
    assert b % block_b == 0

    def body(pref_ref, suf_ref, cls_ref, out_ref):
        out_ref[:, 0:N_PRE, :] = jnp.broadcast_to(
            pref_ref[...], (block_b, N_PRE, CTX_DIM))
        out_ref[:, N_PRE:N_PRE + N_CLS_CTX, :] = cls_ref[...]
        out_ref[:, N_PRE + N_CLS_CTX:TOK_LEN, :] = jnp.broadcast_to(
            suf_ref[...], (block_b, N_SUF, CTX_DIM))

    return pl.pallas_call(
        body,
        grid=(b // block_b,),
        in_specs=[
            pl.BlockSpec((1, N_PRE, CTX_DIM), lambda i: (0, 0, 0)),
            pl.BlockSpec((1, N_SUF, CTX_DIM), lambda i: (0, 0, 0)),
            pl.BlockSpec((block_b, N_CLS_CTX, CTX_DIM), lambda i: (i, 0, 0)),
        ],
        out_specs=pl.BlockSpec((block_b, TOK_LEN, CTX_DIM), lambda i: (i, 0, 0)),
        out_shape=jax.ShapeDtypeStruct((b, TOK_LEN, CTX_DIM), jnp.float32),
    )(token_prefix, token_suffix, cls3)


def kernel(label, cls_ctx, token_prefix, token_suffix):
    cls3 = _sc_gather(cls_ctx, label.astype(jnp.int32))
    return _tc_concat(cls3, token_prefix, token_suffix)
